# baseline (speedup 1.0000x reference)
import functools

import jax
import jax.numpy as jnp
from jax import lax
from jax.experimental import pallas as pl
from jax.experimental.pallas import tpu as pltpu
from jax.experimental.pallas import tpu_sc as plsc

D = 128
B = 4096
S = 200
TIME = 1440
TOTAL = B * S
NC = 2
NS = 16
NW = NC * NS
PER_W = TOTAL // NW
C = 128
CH = PER_W // C
SUPER = 2
NSUP = CH // SUPER
M = 2
LEAD = 1


def _build():
  mesh = plsc.VectorSubcoreMesh(core_axis_name="c", subcore_axis_name="s")

  @functools.partial(
      pl.kernel,
      mesh=mesh,
      out_type=[
          jax.ShapeDtypeStruct((TOTAL, D), jnp.float32),
          jax.ShapeDtypeStruct((TOTAL, D), jnp.float32),
      ],
      scratch_types=[
          pltpu.VMEM((CH, C), jnp.int32),
          pltpu.VMEM_SHARED((TIME, D), jnp.float32),
      ] + [pltpu.VMEM((SUPER * C, D), jnp.float32) for _ in range(M)]
        + [pltpu.SemaphoreType.DMA for _ in range(2 * M)],
  )
  def body(seq_hbm, ts_hbm, tok_hbm, time_hbm, out_tok, out_time,
           idx_v, time_sp, *bufs_and_sems):
    bufs = bufs_and_sems[:M]
    gsems = bufs_and_sems[M:2 * M]
    wsems = bufs_and_sems[2 * M:]
    wid = lax.axis_index("s") * NC + lax.axis_index("c")
    base = wid * PER_W

    @pl.when(lax.axis_index("s") == 0)
    def _():
      pltpu.sync_copy(time_hbm, time_sp)

    def fire(table, k, m):
      for s in range(SUPER):
        pltpu.async_copy(table.at[idx_v.at[k * SUPER + s]],
                         bufs[m].at[pl.ds(s * C, C)], gsems[m])

    def run(idx_hbm, table, out):
      pltpu.sync_copy(idx_hbm.at[wid], idx_v)

      for m in range(LEAD):
        fire(table, m, m)

      def outer(g, carry):
        for m in range(M):
          k = g * M + m
          fk = k + LEAD
          mf = (m + LEAD) % M

          @pl.when((k >= M - LEAD) & (fk < NSUP))
          def _():
            pltpu.make_async_copy(bufs[mf], out.at[pl.ds(0, SUPER * C)],
                                  wsems[mf]).wait()

          @pl.when(fk < NSUP)
          def _():
            fire(table, fk, mf)

          for s in range(SUPER):
            pltpu.make_async_copy(table.at[idx_v.at[0]],
                                  bufs[m].at[pl.ds(s * C, C)],
                                  gsems[m]).wait()
          pltpu.async_copy(bufs[m], out.at[pl.ds(base + k * SUPER * C,
                                                 SUPER * C)], wsems[m])
        return carry

      lax.fori_loop(0, NSUP // M, outer, 0)

      for m in range(M):
        pltpu.make_async_copy(bufs[m], out.at[pl.ds(0, SUPER * C)],
                              wsems[m]).wait()

    with jax.named_scope("tok_phase"):
      run(seq_hbm, tok_hbm, out_tok)
    plsc.subcore_barrier()
    with jax.named_scope("time_phase"):
      run(ts_hbm, time_sp, out_time)

  return body


_gather = _build()


def kernel(seq, ts, token_table, time_table):
  seq3 = seq.astype(jnp.int32).reshape(NW, CH, C)
  ts3 = ts.astype(jnp.int32).reshape(NW, CH, C)
  out_tok, out_time = _gather(seq3, ts3, token_table, time_table)
  return (out_tok.reshape(B, S, D), out_time.reshape(B, S, D))
